# baseline (speedup 1.0000x reference)
import jax
import jax.numpy as jnp
from jax import lax
from jax.experimental import pallas as pl

ALPHA = 16.0


def _fused_body(ids_ref, x_ref, w_ref, b_ref, af_ref, bf_ref, o_ref, *, r, er):
    x = x_ref[...]
    base = jnp.dot(x, w_ref[...], preferred_element_type=jnp.float32)
    lr = jnp.dot(x, af_ref[...], preferred_element_type=jnp.float32)
    ids = ids_ref[...]
    col_expert = lax.broadcasted_iota(jnp.int32, lr.shape, 1) // r
    mask = (col_expert == ids).astype(jnp.float32)
    lr_masked = (lr * mask).astype(jnp.bfloat16)
    delta = jnp.dot(lr_masked, bf_ref[...], preferred_element_type=jnp.float32)
    o_ref[...] = base + b_ref[...] + delta * (ALPHA / r)


def kernel(hidden_states, base_weight, base_bias, lora_a, lora_b, adapter_ids):
    n, d_in = hidden_states.shape
    d_out = base_weight.shape[1]
    e, _, r = lora_a.shape
    er = e * r

    x16 = hidden_states.astype(jnp.bfloat16)
    w16 = base_weight.astype(jnp.bfloat16)
    a_flat = jnp.transpose(lora_a, (1, 0, 2)).reshape(d_in, er).astype(jnp.bfloat16)
    b_flat = lora_b.reshape(er, d_out).astype(jnp.bfloat16)
    ids2d = adapter_ids.astype(jnp.int32).reshape(n, 1)
    bias2d = base_bias.reshape(1, d_out)

    tm = 512
    grid = (n // tm,)

    import functools
    body = functools.partial(_fused_body, r=r, er=er)
    return pl.pallas_call(
        body,
        grid=grid,
        in_specs=[
            pl.BlockSpec((tm, 1), lambda i: (i, 0)),
            pl.BlockSpec((tm, d_in), lambda i: (i, 0)),
            pl.BlockSpec((d_in, d_out), lambda i: (0, 0)),
            pl.BlockSpec((1, d_out), lambda i: (0, 0)),
            pl.BlockSpec((d_in, er), lambda i: (0, 0)),
            pl.BlockSpec((er, d_out), lambda i: (0, 0)),
        ],
        out_specs=pl.BlockSpec((tm, d_out), lambda i: (i, 0)),
        out_shape=jax.ShapeDtypeStruct((n, d_out), jnp.float32),
    )(ids2d, x16, w16, bias2d, a_flat, b_flat)


# speedup vs baseline: 1.5299x; 1.5299x over previous
import jax
import jax.numpy as jnp
from jax import lax
from jax.experimental import pallas as pl

ALPHA = 16.0


def _fused_body(ids_ref, x_ref, w_ref, b_ref, af_ref, bf_ref, o_ref, *, r, er):
    x = x_ref[...]
    base = jnp.dot(x, w_ref[...], preferred_element_type=jnp.float32)
    lr = jnp.dot(x, af_ref[...], preferred_element_type=jnp.float32)
    ids = ids_ref[...]
    col_expert = lax.broadcasted_iota(jnp.int32, lr.shape, 1) // r
    mask = (col_expert == ids).astype(jnp.float32)
    delta = jnp.dot(lr * mask, bf_ref[...], preferred_element_type=jnp.float32)
    o_ref[...] = base + b_ref[...] + delta * (ALPHA / r)


def kernel(hidden_states, base_weight, base_bias, lora_a, lora_b, adapter_ids):
    n, d_in = hidden_states.shape
    d_out = base_weight.shape[1]
    e, _, r = lora_a.shape
    er = e * r

    a_flat = jnp.transpose(lora_a, (1, 0, 2)).reshape(d_in, er)
    b_flat = lora_b.reshape(er, d_out)
    ids2d = adapter_ids.astype(jnp.int32).reshape(n, 1)
    bias2d = base_bias.reshape(1, d_out)

    tm = 1024
    grid = (n // tm,)

    import functools
    body = functools.partial(_fused_body, r=r, er=er)
    return pl.pallas_call(
        body,
        grid=grid,
        in_specs=[
            pl.BlockSpec((tm, 1), lambda i: (i, 0)),
            pl.BlockSpec((tm, d_in), lambda i: (i, 0)),
            pl.BlockSpec((d_in, d_out), lambda i: (0, 0)),
            pl.BlockSpec((1, d_out), lambda i: (0, 0)),
            pl.BlockSpec((d_in, er), lambda i: (0, 0)),
            pl.BlockSpec((er, d_out), lambda i: (0, 0)),
        ],
        out_specs=pl.BlockSpec((tm, d_out), lambda i: (i, 0)),
        out_shape=jax.ShapeDtypeStruct((n, d_out), jnp.float32),
    )(ids2d, hidden_states, base_weight, bias2d, a_flat, b_flat)
